# baseline (speedup 1.0000x reference)
import dataclasses
import functools

import jax
import jax.numpy as jnp
from jax import lax
from jax.experimental import pallas as pl
from jax.experimental.pallas import tpu as pltpu
from jax.experimental.pallas import tpu_sc as plsc

N = 320000
D = 128
G = 512

SC_CORES = 2
SC_SUBCORES = 16
L = 16
NW = SC_CORES * SC_SUBCORES

YW = 128
_ROWS = 16384
GRID = (N + _ROWS - 1) // _ROWS
N_PAD = GRID * _ROWS
Y_ROWS_TOTAL = N_PAD // YW
BLOCK_Y_ROWS = _ROWS // YW

CHUNK_STEPS = (6, 6, 6, 2)
assert sum(CHUNK_STEPS) == GRID


def _matvec_body(block_off, x_ref, w_ref, y_ref):
    i = pl.program_id(0)
    x = x_ref[...]
    w = w_ref[...]
    prod = x * w
    y2 = jnp.sum(prod.reshape(_ROWS // YW, YW, D), axis=2)
    flat = (
        (block_off + i) * _ROWS
        + jax.lax.broadcasted_iota(jnp.int32, y2.shape, 0) * YW
        + jax.lax.broadcasted_iota(jnp.int32, y2.shape, 1)
    )
    y_ref[...] = jnp.where(flat < N, y2, 0.0)


def _matvec_chunk(x, w, block_off, steps):
    return pl.pallas_call(
        functools.partial(_matvec_body, block_off),
        grid=(steps,),
        in_specs=[
            pl.BlockSpec((_ROWS, D), lambda i: (i + block_off, 0)),
            pl.BlockSpec((1, D), lambda i: (0, 0)),
        ],
        out_specs=pl.BlockSpec((_ROWS // YW, YW), lambda i: (i, 0)),
        out_shape=jax.ShapeDtypeStruct((steps * BLOCK_Y_ROWS, YW), jnp.float32),
    )(x, w)


_SC_MESH = plsc.VectorSubcoreMesh(
    core_axis_name="c", subcore_axis_name="s",
    num_cores=SC_CORES, num_subcores=SC_SUBCORES,
)

_SC_PARAMS = pltpu.CompilerParams()
if "needs_layout_passes" in pltpu.CompilerParams.__dataclass_fields__:
    _SC_PARAMS = dataclasses.replace(_SC_PARAMS, needs_layout_passes=False)


def _segsum_body(chunk_row_off, rows_pw, ids_hbm, y_hbm, out_hbm, ids_v, y_v,
                 acc_v, part_v, sem):
    wid = lax.axis_index("s") * SC_CORES + lax.axis_index("c")
    base_row = wid * rows_pw
    cp_ids = pltpu.async_copy(
        ids_hbm.at[pl.ds(chunk_row_off + base_row, rows_pw)], ids_v, sem)
    cp_y = pltpu.async_copy(y_hbm.at[pl.ds(base_row, rows_pw)], y_v, sem)

    zeros = jnp.zeros((L,), jnp.float32)

    @pl.loop(0, L)
    def _zero_row(r):
        for c in range(0, G, L):
            acc_v[r, pl.ds(c, L)] = zeros

    cp_ids.wait()
    cp_y.wait()

    lane = lax.iota(jnp.int32, L)

    @pl.loop(0, rows_pw)
    def _accum_row(r):
        for j in range(0, YW, L):
            plsc.addupdate_scatter(
                acc_v, [lane, ids_v[r, pl.ds(j, L)]], y_v[r, pl.ds(j, L)])

    @pl.loop(0, G, step=L)
    def _fold_col(c):
        s = acc_v[0, pl.ds(c, L)]
        for r in range(1, L):
            s = s + acc_v[r, pl.ds(c, L)]
        part_v[pl.ds(c, L)] = s

    pltpu.sync_copy(part_v, out_hbm.at[wid])


def _make_segsum(chunk_row_off, chunk_y_rows):
    rows_pw = chunk_y_rows // NW
    return functools.partial(
        pl.kernel,
        out_type=jax.ShapeDtypeStruct((NW, G), jnp.float32),
        mesh=_SC_MESH,
        compiler_params=_SC_PARAMS,
        scratch_types=[
            pltpu.VMEM((rows_pw, YW), jnp.int32),
            pltpu.VMEM((rows_pw, YW), jnp.float32),
            pltpu.VMEM((L, G), jnp.float32),
            pltpu.VMEM((G,), jnp.float32),
            pltpu.SemaphoreType.DMA,
        ],
    )(functools.partial(_segsum_body, chunk_row_off, rows_pw))


def _fold_body(pa_ref, pb_ref, pc_ref, pd_ref, b_ref, o_ref):
    s = jnp.sum(pa_ref[...], axis=0, keepdims=True)
    s = s + jnp.sum(pb_ref[...], axis=0, keepdims=True)
    s = s + jnp.sum(pc_ref[...], axis=0, keepdims=True)
    s = s + jnp.sum(pd_ref[...], axis=0, keepdims=True)
    o_ref[...] = s + b_ref[0, 0]


def _fold(parts, b):
    return pl.pallas_call(
        _fold_body,
        in_specs=[pl.BlockSpec((NW, G), lambda: (0, 0)) for _ in parts]
        + [pl.BlockSpec((1, 1), lambda: (0, 0))],
        out_specs=pl.BlockSpec((1, G), lambda: (0, 0)),
        out_shape=jax.ShapeDtypeStruct((1, G), jnp.float32),
    )(*parts, b)


def kernel(cell_features, cell_batches, W, b):
    ids_pad = jnp.concatenate(
        [cell_batches, jnp.zeros((N_PAD - N,), jnp.int32)]
    ).reshape(Y_ROWS_TOTAL, YW)
    parts = []
    block_off = 0
    for steps in CHUNK_STEPS:
        y_c = _matvec_chunk(cell_features, W, block_off, steps)
        segsum = _make_segsum(block_off * BLOCK_Y_ROWS, steps * BLOCK_Y_ROWS)
        parts.append(segsum(ids_pad, y_c))
        block_off += steps
    out = _fold(parts, b.reshape(1, 1))
    return out.reshape(G)


# speedup vs baseline: 1.0266x; 1.0266x over previous
import dataclasses
import functools

import jax
import jax.numpy as jnp
from jax import lax
from jax.experimental import pallas as pl
from jax.experimental.pallas import tpu as pltpu
from jax.experimental.pallas import tpu_sc as plsc

N = 320000
D = 128
G = 512

SC_CORES = 2
SC_SUBCORES = 16
L = 16
NW = SC_CORES * SC_SUBCORES

YW = 128
_ROWS = 16384
GRID = (N + _ROWS - 1) // _ROWS
N_PAD = GRID * _ROWS
Y_ROWS_TOTAL = N_PAD // YW
BLOCK_Y_ROWS = _ROWS // YW

CHUNK_STEPS = (6, 6, 6, 2)
assert sum(CHUNK_STEPS) == GRID


def _matvec_body(block_off, x_ref, w_ref, y_ref):
    i = pl.program_id(0)
    x = x_ref[...]
    w = w_ref[...]
    prod = x * w
    y2 = jnp.sum(prod.reshape(_ROWS // YW, YW, D), axis=2)
    flat = (
        (block_off + i) * _ROWS
        + jax.lax.broadcasted_iota(jnp.int32, y2.shape, 0) * YW
        + jax.lax.broadcasted_iota(jnp.int32, y2.shape, 1)
    )
    y_ref[...] = jnp.where(flat < N, y2, 0.0)


def _matvec_chunk(x, w, block_off, steps):
    return pl.pallas_call(
        functools.partial(_matvec_body, block_off),
        grid=(steps,),
        in_specs=[
            pl.BlockSpec((_ROWS, D), lambda i: (i + block_off, 0)),
            pl.BlockSpec((1, D), lambda i: (0, 0)),
        ],
        out_specs=pl.BlockSpec((_ROWS // YW, YW), lambda i: (i, 0)),
        out_shape=jax.ShapeDtypeStruct((steps * BLOCK_Y_ROWS, YW), jnp.float32),
    )(x, w)


_SC_MESH = plsc.VectorSubcoreMesh(
    core_axis_name="c", subcore_axis_name="s",
    num_cores=SC_CORES, num_subcores=SC_SUBCORES,
)

_SC_PARAMS = pltpu.CompilerParams()
if "needs_layout_passes" in pltpu.CompilerParams.__dataclass_fields__:
    _SC_PARAMS = dataclasses.replace(_SC_PARAMS, needs_layout_passes=False)


def _segsum_body(chunk_row_off, rows_pw, ids_hbm, y_hbm, out_hbm, ids_v, y_v,
                 acc_v, part_v, sem):
    wid = lax.axis_index("s") * SC_CORES + lax.axis_index("c")
    base_row = wid * rows_pw
    cp_ids = pltpu.async_copy(
        ids_hbm.at[pl.ds(chunk_row_off + base_row, rows_pw)], ids_v, sem)
    cp_y = pltpu.async_copy(y_hbm.at[pl.ds(base_row, rows_pw)], y_v, sem)

    zeros = jnp.zeros((L,), jnp.float32)

    @pl.loop(0, L)
    def _zero_row(r):
        for c in range(0, G, L):
            acc_v[r, pl.ds(c, L)] = zeros

    cp_ids.wait()
    cp_y.wait()

    lane = lax.iota(jnp.int32, L)

    @pl.loop(0, rows_pw)
    def _accum_row(r):
        for j in range(0, YW, L):
            plsc.addupdate_scatter(
                acc_v, [lane, ids_v[r, pl.ds(j, L)]], y_v[r, pl.ds(j, L)])

    @pl.loop(0, G, step=L)
    def _fold_col(c):
        s = acc_v[0, pl.ds(c, L)]
        for r in range(1, L):
            s = s + acc_v[r, pl.ds(c, L)]
        part_v[pl.ds(c, L)] = s

    pltpu.sync_copy(part_v, out_hbm.at[wid])


def _make_segsum(chunk_row_off, chunk_y_rows):
    rows_pw = chunk_y_rows // NW
    return functools.partial(
        pl.kernel,
        out_type=jax.ShapeDtypeStruct((NW, G), jnp.float32),
        mesh=_SC_MESH,
        compiler_params=_SC_PARAMS,
        scratch_types=[
            pltpu.VMEM((rows_pw, YW), jnp.int32),
            pltpu.VMEM((rows_pw, YW), jnp.float32),
            pltpu.VMEM((L, G), jnp.float32),
            pltpu.VMEM((G,), jnp.float32),
            pltpu.SemaphoreType.DMA,
        ],
    )(functools.partial(_segsum_body, chunk_row_off, rows_pw))


def _fold_body(pa_ref, pb_ref, pc_ref, pd_ref, b_ref, o_ref):
    s = jnp.sum(pa_ref[...], axis=0, keepdims=True)
    s = s + jnp.sum(pb_ref[...], axis=0, keepdims=True)
    s = s + jnp.sum(pc_ref[...], axis=0, keepdims=True)
    s = s + jnp.sum(pd_ref[...], axis=0, keepdims=True)
    o_ref[...] = s + b_ref[0, 0]


def _fold(parts, b):
    return pl.pallas_call(
        _fold_body,
        in_specs=[pl.BlockSpec((NW, G), lambda: (0, 0)) for _ in parts]
        + [pl.BlockSpec((1, 1), lambda: (0, 0))],
        out_specs=pl.BlockSpec((1, G), lambda: (0, 0)),
        out_shape=jax.ShapeDtypeStruct((1, G), jnp.float32),
    )(*parts, b)


def kernel(cell_features, cell_batches, W, b):
    ids_raw = cell_batches.reshape(N // YW, YW)
    ids_pad = jnp.concatenate(
        [cell_batches, jnp.zeros((N_PAD - N,), jnp.int32)]
    ).reshape(Y_ROWS_TOTAL, YW)
    parts = []
    block_off = 0
    for steps in CHUNK_STEPS:
        row_off = block_off * BLOCK_Y_ROWS
        rows = steps * BLOCK_Y_ROWS
        ids_src = ids_raw if (row_off + rows) <= N // YW else ids_pad
        y_c = _matvec_chunk(cell_features, W, block_off, steps)
        parts.append(_make_segsum(row_off, rows)(ids_src, y_c))
        block_off += steps
    out = _fold(parts, b.reshape(1, 1))
    return out.reshape(G)
